# Initial kernel scaffold; baseline (speedup 1.0000x reference)
import jax
import jax.numpy as jnp
from jax.experimental import pallas as pl


def kernel(x, edge_index, edge_attr, Wq, bq, Wk, bk, Wv, bv, We, be, Wo, bo):
    raise NotImplementedError("write your pallas kernel here")


# speedup vs baseline: 2.2450x; 2.2450x over previous
import functools

import jax
import jax.numpy as jnp
from jax import lax
from jax.experimental import pallas as pl
from jax.experimental.pallas import tpu as pltpu
from jax.experimental.pallas import tpu_sc as plsc

N = 10000
E = 320000
DF = 128
H = 8
D = 16
HD = H * D

NP = 10240
ROWS_PER_TILE = NP // 16
C = 128
NCHUNK = E // C
NW = 32
TRIPS = -(-NCHUNK // NW)


def _proj_body(x_ref, wq_ref, bq_ref, wk_ref, bk_ref, wv_ref, bv_ref,
               q_ref, k_ref, v_ref):
    xb = x_ref[...]
    hi = jax.lax.Precision.HIGHEST
    q_ref[...] = jnp.dot(xb, wq_ref[...], precision=hi) + bq_ref[...]
    k_ref[...] = jnp.dot(xb, wk_ref[...], precision=hi) + bk_ref[...]
    v_ref[...] = jnp.dot(xb, wv_ref[...], precision=hi) + bv_ref[...]


def _proj(x, wq, bq, wk, bk, wv2, bv2):
    blk = 2000
    grid = N // blk
    wspec = pl.BlockSpec((DF, HD), lambda i: (0, 0))
    bspec = pl.BlockSpec((1, HD), lambda i: (0, 0))
    return pl.pallas_call(
        _proj_body,
        grid=(grid,),
        in_specs=[pl.BlockSpec((blk, DF), lambda i: (i, 0)),
                  wspec, bspec, wspec, bspec, wspec, bspec],
        out_specs=[pl.BlockSpec((blk, HD), lambda i: (i, 0)),
                   pl.BlockSpec((blk, HD), lambda i: (i, 0)),
                   pl.BlockSpec((blk, HD), lambda i: (i, 0))],
        out_shape=[jax.ShapeDtypeStruct((N, HD), jnp.float32),
                   jax.ShapeDtypeStruct((N, HD), jnp.float32),
                   jax.ShapeDtypeStruct((N, HD), jnp.float32)],
    )(x, wq, bq.reshape(1, HD), wk, bk.reshape(1, HD), wv2,
      bv2.reshape(1, HD))


def _esq_body(ea_ref, we_ref, be_ref, sel_ref, o_ref):
    hi = jax.lax.Precision.HIGHEST
    t = jnp.dot(ea_ref[...], we_ref[...], precision=hi) + be_ref[...]
    o_ref[...] = jnp.dot(t * t, sel_ref[...], precision=hi)


def _esq(edge_attr, we, be, sel):
    blk = 8000
    grid = E // blk
    de = edge_attr.shape[1]
    return pl.pallas_call(
        _esq_body,
        grid=(grid,),
        in_specs=[pl.BlockSpec((blk, de), lambda i: (i, 0)),
                  pl.BlockSpec((de, HD), lambda i: (0, 0)),
                  pl.BlockSpec((1, HD), lambda i: (0, 0)),
                  pl.BlockSpec((HD, H), lambda i: (0, 0))],
        out_specs=pl.BlockSpec((blk, H), lambda i: (i, 0)),
        out_shape=jax.ShapeDtypeStruct((E, H), jnp.float32),
    )(edge_attr, we, be.reshape(1, HD), sel)


NBLK = E // 16
W = 128
BTRIPS = NBLK // NW


def _sc_edges_body(src_hbm, dst_hbm, q_hbm, k_hbm, v_hbm, esq_hbm,
                   part_hbm,
                   sidx, didx, qb, kb, vb, eb, ob, acc, sem0):
    c = lax.axis_index("c")
    s = lax.axis_index("s")
    w = s * 2 + c

    for i in range(16):
        for t in range(W // 16):
            ob[i, pl.ds(t * 16, 16)] = jnp.zeros((16,), jnp.float32)

    row0 = s * ROWS_PER_TILE
    for j in range(ROWS_PER_TILE // 16):
        pltpu.sync_copy(ob, acc.at[pl.ds(row0 + j * 16, 16)])

    for i in range(16):
        ob[i, pl.ds(16, 16)] = (
            lax.iota(jnp.int32, 16) == 0).astype(jnp.float32)

    plsc.subcore_barrier()

    rows = lax.iota(jnp.int32, 16)
    rows_h = rows * H

    @pl.loop(0, BTRIPS)
    def _blk(i):
        ebase = (i * NW + w) * 16
        pltpu.sync_copy(src_hbm.at[pl.ds(ebase, 16)], sidx)
        pltpu.sync_copy(dst_hbm.at[pl.ds(ebase, 16)], didx)
        pltpu.sync_copy(esq_hbm.at[pl.ds(ebase * H, 16 * H)], eb)
        sv = sidx[...]
        dv = didx[...]
        pltpu.async_copy(q_hbm.at[sv], qb, sem0).wait()
        pltpu.async_copy(k_hbm.at[dv], kb, sem0).wait()
        pltpu.async_copy(v_hbm.at[dv], vb, sem0).wait()

        col = lambda f: jnp.full((16,), f, jnp.int32)
        accs = [plsc.load_gather(eb, [rows_h + h]) for h in range(H)]
        for f in range(HD):
            qv = plsc.load_gather(qb, [rows, col(f)])
            kv = plsc.load_gather(kb, [rows, col(f)])
            accs[f // D] = accs[f // D] + qv * kv
        m = jnp.maximum(jnp.maximum(jnp.maximum(accs[0], accs[1]),
                                    jnp.maximum(accs[2], accs[3])),
                        jnp.maximum(jnp.maximum(accs[4], accs[5]),
                                    jnp.maximum(accs[6], accs[7])))
        es = [jnp.exp(a - m) for a in accs]
        tot = (((es[0] + es[1]) + (es[2] + es[3]))
               + ((es[4] + es[5]) + (es[6] + es[7])))
        sca = 1.0 / tot
        al = [e * sca for e in es]
        for j in range(D):
            vv = plsc.load_gather(vb, [rows, col(j)])
            vj = al[0] * vv
            for h in range(1, H):
                vv = plsc.load_gather(vb, [rows, col(h * D + j)])
                vj = vj + al[h] * vv
            plsc.store_scatter(ob, [rows, col(j)], vj)

        pltpu.async_copy(ob, acc.at[sv], sem0, add=True).wait()

    plsc.subcore_barrier()
    out_row = c * NP + s * ROWS_PER_TILE
    for j in range(ROWS_PER_TILE // 16):
        pltpu.sync_copy(acc.at[pl.ds(row0 + j * 16, 16)], ob)
        pltpu.sync_copy(ob, part_hbm.at[pl.ds(out_row + j * 16, 16)])


def _sc_edges(src, dst, q, k, v, esq):
    mesh = plsc.VectorSubcoreMesh(core_axis_name="c", subcore_axis_name="s")
    fn = pl.kernel(
        _sc_edges_body,
        out_type=jax.ShapeDtypeStruct((2 * NP, W), jnp.float32),
        mesh=mesh,
        compiler_params=pltpu.CompilerParams(needs_layout_passes=False),
        scratch_types=[
            pltpu.VMEM((16,), jnp.int32),
            pltpu.VMEM((16,), jnp.int32),
            pltpu.VMEM((16, HD), jnp.float32),
            pltpu.VMEM((16, HD), jnp.float32),
            pltpu.VMEM((16, HD), jnp.float32),
            pltpu.VMEM((16 * H,), jnp.float32),
            pltpu.VMEM((16, W), jnp.float32),
            pltpu.VMEM_SHARED((NP, W), jnp.float32),
            pltpu.SemaphoreType.DMA,
        ],
    )
    return fn(src, dst, q, k, v, esq)


def _combine_body(p0_ref, p1_ref, c0_ref, c1_ref, bo_ref, o_ref):
    csum = c0_ref[...] + c1_ref[...]
    s = p0_ref[...] + p1_ref[...]
    denom = jnp.maximum(csum, 1.0)
    o_ref[...] = s / denom + bo_ref[...] * (csum > 0.0)


def _combine(p0, p1, c0, c1, bo):
    pspec = pl.BlockSpec((N, D), lambda: (0, 0))
    cspec = pl.BlockSpec((N, 1), lambda: (0, 0))
    return pl.pallas_call(
        _combine_body,
        in_specs=[pspec, pspec, cspec, cspec,
                  pl.BlockSpec((1, D), lambda: (0, 0))],
        out_specs=pl.BlockSpec((N, D), lambda: (0, 0)),
        out_shape=jax.ShapeDtypeStruct((N, D), jnp.float32),
    )(p0, p1, c0, c1, bo.reshape(1, D))


@jax.jit
def kernel(x, edge_index, edge_attr, Wq, bq, Wk, bk, Wv, bv, We, be, Wo, bo):
    scale = 1.0 / (D ** 0.5)
    wq = Wq * scale
    bqs = bq * scale
    wv2 = jnp.einsum('fhd,hde->fhe', Wv.reshape(DF, H, D),
                     Wo.reshape(H, D, D)).reshape(DF, HD)
    bv2 = jnp.einsum('hd,hde->he', bv.reshape(H, D),
                     Wo.reshape(H, D, D)).reshape(HD)
    sel = (jnp.kron(jnp.eye(H, dtype=jnp.float32),
                    jnp.ones((D, 1), dtype=jnp.float32)) * scale)

    q, k, v = _proj(x, wq, bqs, Wk, bk, wv2, bv2)
    esq = _esq(edge_attr, We, be, sel).reshape(E * H)

    src = edge_index[0]
    dst = edge_index[1]
    part = _sc_edges(src, dst, q, k, v, esq)

    p0 = part[:N, :D]
    p1 = part[NP:NP + N, :D]
    c0 = part[:N, D:D + 1]
    c1 = part[NP:NP + N, D:D + 1]
    return _combine(p0, p1, c0, c1, bo)


# speedup vs baseline: 2.6238x; 1.1688x over previous
import functools

import jax
import jax.numpy as jnp
from jax import lax
from jax.experimental import pallas as pl
from jax.experimental.pallas import tpu as pltpu
from jax.experimental.pallas import tpu_sc as plsc

N = 10000
E = 320000
DF = 128
H = 8
D = 16
HD = H * D

NP = 10240
ROWS_PER_TILE = NP // 16
C = 128
NCHUNK = E // C
NW = 32
TRIPS = -(-NCHUNK // NW)


def _proj_body(x_ref, wq_ref, bq_ref, wk_ref, bk_ref, wv_ref, bv_ref,
               q_ref, k_ref, v_ref):
    xb = x_ref[...]
    hi = jax.lax.Precision.HIGHEST
    q_ref[...] = jnp.dot(xb, wq_ref[...], precision=hi) + bq_ref[...]
    k_ref[...] = jnp.dot(xb, wk_ref[...], precision=hi) + bk_ref[...]
    v_ref[...] = jnp.dot(xb, wv_ref[...], precision=hi) + bv_ref[...]


def _proj(x, wq, bq, wk, bk, wv2, bv2):
    blk = 2000
    grid = N // blk
    wspec = pl.BlockSpec((DF, HD), lambda i: (0, 0))
    bspec = pl.BlockSpec((1, HD), lambda i: (0, 0))
    return pl.pallas_call(
        _proj_body,
        grid=(grid,),
        in_specs=[pl.BlockSpec((blk, DF), lambda i: (i, 0)),
                  wspec, bspec, wspec, bspec, wspec, bspec],
        out_specs=[pl.BlockSpec((blk, HD), lambda i: (i, 0)),
                   pl.BlockSpec((blk, HD), lambda i: (i, 0)),
                   pl.BlockSpec((blk, HD), lambda i: (i, 0))],
        out_shape=[jax.ShapeDtypeStruct((N, HD), jnp.float32),
                   jax.ShapeDtypeStruct((N, HD), jnp.float32),
                   jax.ShapeDtypeStruct((N, HD), jnp.float32)],
    )(x, wq, bq.reshape(1, HD), wk, bk.reshape(1, HD), wv2,
      bv2.reshape(1, HD))


def _esq_body(ea_ref, we_ref, be_ref, sel_ref, o_ref):
    hi = jax.lax.Precision.HIGHEST
    t = jnp.dot(ea_ref[...], we_ref[...], precision=hi) + be_ref[...]
    o_ref[...] = jnp.dot(t * t, sel_ref[...], precision=hi)


def _esq(edge_attr, we, be, sel):
    blk = 8000
    grid = E // blk
    de = edge_attr.shape[1]
    return pl.pallas_call(
        _esq_body,
        grid=(grid,),
        in_specs=[pl.BlockSpec((blk, de), lambda i: (i, 0)),
                  pl.BlockSpec((de, HD), lambda i: (0, 0)),
                  pl.BlockSpec((1, HD), lambda i: (0, 0)),
                  pl.BlockSpec((HD, H), lambda i: (0, 0))],
        out_specs=pl.BlockSpec((blk, H), lambda i: (i, 0)),
        out_shape=jax.ShapeDtypeStruct((E, H), jnp.float32),
    )(edge_attr, we, be.reshape(1, HD), sel)


NBLK = E // 16
W = 128
BTRIPS = NBLK // NW


def _sc_edges_body(src_hbm, dst_hbm, q_hbm, k_hbm, v_hbm, esq_hbm,
                   part_hbm,
                   sidx, didx, qb, kb, vb, eb, ob, acc, sem0, sem1, sem2):
    c = lax.axis_index("c")
    s = lax.axis_index("s")
    w = s * 2 + c

    for i in range(16):
        for t in range(W // 16):
            ob[i, pl.ds(t * 16, 16)] = jnp.zeros((16,), jnp.float32)

    row0 = s * ROWS_PER_TILE
    for j in range(ROWS_PER_TILE // 16):
        pltpu.sync_copy(ob, acc.at[pl.ds(row0 + j * 16, 16)])

    for i in range(16):
        ob[i, pl.ds(16, 16)] = (
            lax.iota(jnp.int32, 16) == 0).astype(jnp.float32)

    plsc.subcore_barrier()

    rows = lax.iota(jnp.int32, 16)
    rows_h = rows * H

    @pl.loop(0, BTRIPS)
    def _blk(i):
        ebase = (i * NW + w) * 16
        pltpu.sync_copy(src_hbm.at[pl.ds(ebase, 16)], sidx)
        pltpu.sync_copy(dst_hbm.at[pl.ds(ebase, 16)], didx)
        pltpu.sync_copy(esq_hbm.at[pl.ds(ebase * H, 16 * H)], eb)
        sv = sidx[...]
        dv = didx[...]
        cp_q = pltpu.async_copy(q_hbm.at[sv], qb, sem0)
        cp_k = pltpu.async_copy(k_hbm.at[dv], kb, sem1)
        cp_v = pltpu.async_copy(v_hbm.at[dv], vb, sem2)
        cp_q.wait()
        cp_k.wait()
        cp_v.wait()

        col = lambda f: jnp.full((16,), f, jnp.int32)
        accs = [plsc.load_gather(eb, [rows_h + h]) for h in range(H)]
        for f in range(HD):
            qv = plsc.load_gather(qb, [rows, col(f)])
            kv = plsc.load_gather(kb, [rows, col(f)])
            accs[f // D] = accs[f // D] + qv * kv
        m = jnp.maximum(jnp.maximum(jnp.maximum(accs[0], accs[1]),
                                    jnp.maximum(accs[2], accs[3])),
                        jnp.maximum(jnp.maximum(accs[4], accs[5]),
                                    jnp.maximum(accs[6], accs[7])))
        es = [jnp.exp(a - m) for a in accs]
        tot = (((es[0] + es[1]) + (es[2] + es[3]))
               + ((es[4] + es[5]) + (es[6] + es[7])))
        sca = 1.0 / tot
        al = [e * sca for e in es]
        for j in range(D):
            vv = plsc.load_gather(vb, [rows, col(j)])
            vj = al[0] * vv
            for h in range(1, H):
                vv = plsc.load_gather(vb, [rows, col(h * D + j)])
                vj = vj + al[h] * vv
            plsc.store_scatter(ob, [rows, col(j)], vj)

        pltpu.async_copy(ob, acc.at[sv], sem0, add=True).wait()

    plsc.subcore_barrier()
    out_row = c * NP + s * ROWS_PER_TILE
    for j in range(ROWS_PER_TILE // 16):
        pltpu.sync_copy(acc.at[pl.ds(row0 + j * 16, 16)], ob)
        pltpu.sync_copy(ob, part_hbm.at[pl.ds(out_row + j * 16, 16)])


def _sc_edges(src, dst, q, k, v, esq):
    mesh = plsc.VectorSubcoreMesh(core_axis_name="c", subcore_axis_name="s")
    fn = pl.kernel(
        _sc_edges_body,
        out_type=jax.ShapeDtypeStruct((2 * NP, W), jnp.float32),
        mesh=mesh,
        compiler_params=pltpu.CompilerParams(needs_layout_passes=False),
        scratch_types=[
            pltpu.VMEM((16,), jnp.int32),
            pltpu.VMEM((16,), jnp.int32),
            pltpu.VMEM((16, HD), jnp.float32),
            pltpu.VMEM((16, HD), jnp.float32),
            pltpu.VMEM((16, HD), jnp.float32),
            pltpu.VMEM((16 * H,), jnp.float32),
            pltpu.VMEM((16, W), jnp.float32),
            pltpu.VMEM_SHARED((NP, W), jnp.float32),
            pltpu.SemaphoreType.DMA,
            pltpu.SemaphoreType.DMA,
            pltpu.SemaphoreType.DMA,
        ],
    )
    return fn(src, dst, q, k, v, esq)


def _combine_body(p0_ref, p1_ref, c0_ref, c1_ref, bo_ref, o_ref):
    csum = c0_ref[...] + c1_ref[...]
    s = p0_ref[...] + p1_ref[...]
    denom = jnp.maximum(csum, 1.0)
    o_ref[...] = s / denom + bo_ref[...] * (csum > 0.0)


def _combine(p0, p1, c0, c1, bo):
    pspec = pl.BlockSpec((N, D), lambda: (0, 0))
    cspec = pl.BlockSpec((N, 1), lambda: (0, 0))
    return pl.pallas_call(
        _combine_body,
        in_specs=[pspec, pspec, cspec, cspec,
                  pl.BlockSpec((1, D), lambda: (0, 0))],
        out_specs=pl.BlockSpec((N, D), lambda: (0, 0)),
        out_shape=jax.ShapeDtypeStruct((N, D), jnp.float32),
    )(p0, p1, c0, c1, bo.reshape(1, D))


@jax.jit
def kernel(x, edge_index, edge_attr, Wq, bq, Wk, bk, Wv, bv, We, be, Wo, bo):
    scale = 1.0 / (D ** 0.5)
    wq = Wq * scale
    bqs = bq * scale
    wv2 = jnp.einsum('fhd,hde->fhe', Wv.reshape(DF, H, D),
                     Wo.reshape(H, D, D)).reshape(DF, HD)
    bv2 = jnp.einsum('hd,hde->he', bv.reshape(H, D),
                     Wo.reshape(H, D, D)).reshape(HD)
    sel = (jnp.kron(jnp.eye(H, dtype=jnp.float32),
                    jnp.ones((D, 1), dtype=jnp.float32)) * scale)

    q, k, v = _proj(x, wq, bqs, Wk, bk, wv2, bv2)
    esq = _esq(edge_attr, We, be, sel).reshape(E * H)

    src = edge_index[0]
    dst = edge_index[1]
    part = _sc_edges(src, dst, q, k, v, esq)

    p0 = part[:N, :D]
    p1 = part[NP:NP + N, :D]
    c0 = part[:N, D:D + 1]
    c1 = part[NP:NP + N, D:D + 1]
    return _combine(p0, p1, c0, c1, bo)


# speedup vs baseline: 2.9374x; 1.1195x over previous
import functools

import jax
import jax.numpy as jnp
from jax import lax
from jax.experimental import pallas as pl
from jax.experimental.pallas import tpu as pltpu
from jax.experimental.pallas import tpu_sc as plsc

N = 10000
E = 320000
DF = 128
H = 8
D = 16
HD = H * D

NP = 10240
ROWS_PER_TILE = NP // 16
C = 128
NCHUNK = E // C
NW = 32
TRIPS = -(-NCHUNK // NW)


def _proj_body(x_ref, wq_ref, bq_ref, wk_ref, bk_ref, wv_ref, bv_ref,
               q_ref, k_ref, v_ref):
    xb = x_ref[...]
    hi = jax.lax.Precision.HIGHEST
    q_ref[...] = jnp.dot(xb, wq_ref[...], precision=hi) + bq_ref[...]
    k_ref[...] = jnp.dot(xb, wk_ref[...], precision=hi) + bk_ref[...]
    v_ref[...] = jnp.dot(xb, wv_ref[...], precision=hi) + bv_ref[...]


def _proj(x, wq, bq, wk, bk, wv2, bv2):
    blk = 2000
    grid = N // blk
    wspec = pl.BlockSpec((DF, HD), lambda i: (0, 0))
    bspec = pl.BlockSpec((1, HD), lambda i: (0, 0))
    return pl.pallas_call(
        _proj_body,
        grid=(grid,),
        in_specs=[pl.BlockSpec((blk, DF), lambda i: (i, 0)),
                  wspec, bspec, wspec, bspec, wspec, bspec],
        out_specs=[pl.BlockSpec((blk, HD), lambda i: (i, 0)),
                   pl.BlockSpec((blk, HD), lambda i: (i, 0)),
                   pl.BlockSpec((blk, HD), lambda i: (i, 0))],
        out_shape=[jax.ShapeDtypeStruct((N, HD), jnp.float32),
                   jax.ShapeDtypeStruct((N, HD), jnp.float32),
                   jax.ShapeDtypeStruct((N, HD), jnp.float32)],
    )(x, wq, bq.reshape(1, HD), wk, bk.reshape(1, HD), wv2,
      bv2.reshape(1, HD))


def _esq_body(ea_ref, we_ref, be_ref, sel_ref, o_ref):
    hi = jax.lax.Precision.HIGHEST
    t = jnp.dot(ea_ref[...], we_ref[...], precision=hi) + be_ref[...]
    o_ref[...] = jnp.dot(t * t, sel_ref[...], precision=hi)


def _esq(edge_attr, we, be, sel):
    blk = 8000
    grid = E // blk
    de = edge_attr.shape[1]
    return pl.pallas_call(
        _esq_body,
        grid=(grid,),
        in_specs=[pl.BlockSpec((blk, de), lambda i: (i, 0)),
                  pl.BlockSpec((de, HD), lambda i: (0, 0)),
                  pl.BlockSpec((1, HD), lambda i: (0, 0)),
                  pl.BlockSpec((HD, H), lambda i: (0, 0))],
        out_specs=pl.BlockSpec((blk, H), lambda i: (i, 0)),
        out_shape=jax.ShapeDtypeStruct((E, H), jnp.float32),
    )(edge_attr, we, be.reshape(1, HD), sel)


W = 128
B2 = 32
NBLK = E // B2
BTRIPS = -(-NBLK // NW)


def _sc_edges_body(src_hbm, dst_hbm, q_hbm, k_hbm, v_hbm, esq_hbm,
                   part_hbm,
                   sidx, didx, qb, kb, vb, eb, ob, acc,
                   sem0, sem1, sem2, sem3, sem4, sem5):
    c = lax.axis_index("c")
    s = lax.axis_index("s")
    w = s * 2 + c

    for i in range(B2):
        for t in range(W // 16):
            ob[i, pl.ds(t * 16, 16)] = jnp.zeros((16,), jnp.float32)

    row0 = s * ROWS_PER_TILE
    for j in range(ROWS_PER_TILE // B2):
        pltpu.sync_copy(ob, acc.at[pl.ds(row0 + j * B2, B2)])

    plsc.subcore_barrier()

    rows = lax.iota(jnp.int32, 16)

    def compute(rbase, valf):
        r = rows + rbase
        col = lambda f: jnp.full((16,), f, jnp.int32)
        r_h = r * H
        accs = [plsc.load_gather(eb, [r_h + h]) for h in range(H)]
        for f in range(HD):
            qv = plsc.load_gather(qb, [r, col(f)])
            kv = plsc.load_gather(kb, [r, col(f)])
            accs[f // D] = accs[f // D] + qv * kv
        m = jnp.maximum(jnp.maximum(jnp.maximum(accs[0], accs[1]),
                                    jnp.maximum(accs[2], accs[3])),
                        jnp.maximum(jnp.maximum(accs[4], accs[5]),
                                    jnp.maximum(accs[6], accs[7])))
        es = [jnp.exp(a - m) for a in accs]
        tot = (((es[0] + es[1]) + (es[2] + es[3]))
               + ((es[4] + es[5]) + (es[6] + es[7])))
        sca = valf / tot
        al = [e * sca for e in es]
        for j in range(D):
            vv = plsc.load_gather(vb, [r, col(j)])
            vj = al[0] * vv
            for h in range(1, H):
                vv = plsc.load_gather(vb, [r, col(h * D + j)])
                vj = vj + al[h] * vv
            plsc.store_scatter(ob, [r, col(j)], vj)
        plsc.store_scatter(ob, [r, col(D)],
                           jnp.zeros((16,), jnp.float32) + valf)

    @pl.loop(0, BTRIPS)
    def _blk(i):
        raw = i * NW + w
        blk = jnp.minimum(raw, NBLK - 1)
        valf = (raw < NBLK).astype(jnp.float32)
        ebase = blk * B2
        pltpu.sync_copy(src_hbm.at[pl.ds(ebase, B2)], sidx)
        pltpu.sync_copy(dst_hbm.at[pl.ds(ebase, B2)], didx)
        pltpu.sync_copy(esq_hbm.at[pl.ds(ebase * H, B2 * H)], eb)
        sva = sidx[pl.ds(0, 16)]
        svb = sidx[pl.ds(16, 16)]
        dva = didx[pl.ds(0, 16)]
        dvb = didx[pl.ds(16, 16)]
        cps = [pltpu.async_copy(q_hbm.at[sva], qb.at[pl.ds(0, 16)], sem0),
               pltpu.async_copy(k_hbm.at[dva], kb.at[pl.ds(0, 16)], sem1),
               pltpu.async_copy(v_hbm.at[dva], vb.at[pl.ds(0, 16)], sem2),
               pltpu.async_copy(q_hbm.at[svb], qb.at[pl.ds(16, 16)], sem3),
               pltpu.async_copy(k_hbm.at[dvb], kb.at[pl.ds(16, 16)], sem4),
               pltpu.async_copy(v_hbm.at[dvb], vb.at[pl.ds(16, 16)], sem5)]
        for cp in cps:
            cp.wait()
        compute(0, valf)
        compute(16, valf)
        cpa = pltpu.async_copy(ob.at[pl.ds(0, 16)], acc.at[sva], sem0,
                               add=True)
        cpb = pltpu.async_copy(ob.at[pl.ds(16, 16)], acc.at[svb], sem1,
                               add=True)
        cpa.wait()
        cpb.wait()

    plsc.subcore_barrier()
    out_row = c * NP + s * ROWS_PER_TILE
    for j in range(ROWS_PER_TILE // B2):
        pltpu.sync_copy(acc.at[pl.ds(row0 + j * B2, B2)], ob)
        pltpu.sync_copy(ob, part_hbm.at[pl.ds(out_row + j * B2, B2)])


def _sc_edges(src, dst, q, k, v, esq):
    mesh = plsc.VectorSubcoreMesh(core_axis_name="c", subcore_axis_name="s")
    fn = pl.kernel(
        _sc_edges_body,
        out_type=jax.ShapeDtypeStruct((2 * NP, W), jnp.float32),
        mesh=mesh,
        compiler_params=pltpu.CompilerParams(needs_layout_passes=False),
        scratch_types=[
            pltpu.VMEM((B2,), jnp.int32),
            pltpu.VMEM((B2,), jnp.int32),
            pltpu.VMEM((B2, HD), jnp.float32),
            pltpu.VMEM((B2, HD), jnp.float32),
            pltpu.VMEM((B2, HD), jnp.float32),
            pltpu.VMEM((B2 * H,), jnp.float32),
            pltpu.VMEM((B2, W), jnp.float32),
            pltpu.VMEM_SHARED((NP, W), jnp.float32),
            pltpu.SemaphoreType.DMA,
            pltpu.SemaphoreType.DMA,
            pltpu.SemaphoreType.DMA,
            pltpu.SemaphoreType.DMA,
            pltpu.SemaphoreType.DMA,
            pltpu.SemaphoreType.DMA,
        ],
    )
    return fn(src, dst, q, k, v, esq)


def _combine_body(p0_ref, p1_ref, c0_ref, c1_ref, bo_ref, o_ref):
    csum = c0_ref[...] + c1_ref[...]
    s = p0_ref[...] + p1_ref[...]
    denom = jnp.maximum(csum, 1.0)
    o_ref[...] = s / denom + bo_ref[...] * (csum > 0.0)


def _combine(p0, p1, c0, c1, bo):
    pspec = pl.BlockSpec((N, D), lambda: (0, 0))
    cspec = pl.BlockSpec((N, 1), lambda: (0, 0))
    return pl.pallas_call(
        _combine_body,
        in_specs=[pspec, pspec, cspec, cspec,
                  pl.BlockSpec((1, D), lambda: (0, 0))],
        out_specs=pl.BlockSpec((N, D), lambda: (0, 0)),
        out_shape=jax.ShapeDtypeStruct((N, D), jnp.float32),
    )(p0, p1, c0, c1, bo.reshape(1, D))


@jax.jit
def kernel(x, edge_index, edge_attr, Wq, bq, Wk, bk, Wv, bv, We, be, Wo, bo):
    scale = 1.0 / (D ** 0.5)
    wq = Wq * scale
    bqs = bq * scale
    wv2 = jnp.einsum('fhd,hde->fhe', Wv.reshape(DF, H, D),
                     Wo.reshape(H, D, D)).reshape(DF, HD)
    bv2 = jnp.einsum('hd,hde->he', bv.reshape(H, D),
                     Wo.reshape(H, D, D)).reshape(HD)
    sel = (jnp.kron(jnp.eye(H, dtype=jnp.float32),
                    jnp.ones((D, 1), dtype=jnp.float32)) * scale)

    q, k, v = _proj(x, wq, bqs, Wk, bk, wv2, bv2)
    esq = _esq(edge_attr, We, be, sel).reshape(E * H)

    src = edge_index[0]
    dst = edge_index[1]
    part = _sc_edges(src, dst, q, k, v, esq)

    p0 = part[:N, :D]
    p1 = part[NP:NP + N, :D]
    c0 = part[:N, D:D + 1]
    c1 = part[NP:NP + N, D:D + 1]
    return _combine(p0, p1, c0, c1, bo)


# speedup vs baseline: 3.2693x; 1.1130x over previous
import functools

import jax
import jax.numpy as jnp
from jax import lax
from jax.experimental import pallas as pl
from jax.experimental.pallas import tpu as pltpu
from jax.experimental.pallas import tpu_sc as plsc

N = 10000
E = 320000
DF = 128
H = 8
D = 16
HD = H * D

NP = 10240
ROWS_PER_TILE = NP // 16
C = 128
NCHUNK = E // C
NW = 32
TRIPS = -(-NCHUNK // NW)


def _proj_body(x_ref, wq_ref, bq_ref, wk_ref, bk_ref, wv_ref, bv_ref,
               q_ref, k_ref, v_ref):
    xb = x_ref[...]
    hi = jax.lax.Precision.HIGHEST
    q_ref[...] = jnp.dot(xb, wq_ref[...], precision=hi) + bq_ref[...]
    k_ref[...] = jnp.dot(xb, wk_ref[...], precision=hi) + bk_ref[...]
    v_ref[...] = jnp.dot(xb, wv_ref[...], precision=hi) + bv_ref[...]


def _proj(x, wq, bq, wk, bk, wv2, bv2):
    blk = 2000
    grid = N // blk
    wspec = pl.BlockSpec((DF, HD), lambda i: (0, 0))
    bspec = pl.BlockSpec((1, HD), lambda i: (0, 0))
    return pl.pallas_call(
        _proj_body,
        grid=(grid,),
        in_specs=[pl.BlockSpec((blk, DF), lambda i: (i, 0)),
                  wspec, bspec, wspec, bspec, wspec, bspec],
        out_specs=[pl.BlockSpec((blk, HD), lambda i: (i, 0)),
                   pl.BlockSpec((blk, HD), lambda i: (i, 0)),
                   pl.BlockSpec((blk, HD), lambda i: (i, 0))],
        out_shape=[jax.ShapeDtypeStruct((N, HD), jnp.float32),
                   jax.ShapeDtypeStruct((N, HD), jnp.float32),
                   jax.ShapeDtypeStruct((N, HD), jnp.float32)],
    )(x, wq, bq.reshape(1, HD), wk, bk.reshape(1, HD), wv2,
      bv2.reshape(1, HD))


def _esq_body(ea_ref, we_ref, be_ref, sel_ref, o_ref):
    hi = jax.lax.Precision.HIGHEST
    t = jnp.dot(ea_ref[...], we_ref[...], precision=hi) + be_ref[...]
    o_ref[...] = jnp.dot(t * t, sel_ref[...], precision=hi)


def _esq(edge_attr, we, be, sel):
    blk = 8000
    grid = E // blk
    de = edge_attr.shape[1]
    return pl.pallas_call(
        _esq_body,
        grid=(grid,),
        in_specs=[pl.BlockSpec((blk, de), lambda i: (i, 0)),
                  pl.BlockSpec((de, HD), lambda i: (0, 0)),
                  pl.BlockSpec((1, HD), lambda i: (0, 0)),
                  pl.BlockSpec((HD, H), lambda i: (0, 0))],
        out_specs=pl.BlockSpec((blk, H), lambda i: (i, 0)),
        out_shape=jax.ShapeDtypeStruct((E, H), jnp.float32),
    )(edge_attr, we, be.reshape(1, HD), sel)


W = 128
B2 = 64
NBLK = E // B2
BTRIPS = -(-NBLK // NW)


def _sc_edges_body(src_hbm, dst_hbm, q_hbm, k_hbm, v_hbm, esq_hbm,
                   part_hbm,
                   sidx, didx, qb, kb, vb, eb, ob, acc,
                   sem0, sem1, sem2, sem3, sem4, sem5):
    c = lax.axis_index("c")
    s = lax.axis_index("s")
    w = s * 2 + c

    for i in range(B2):
        for t in range(W // 16):
            ob[i, pl.ds(t * 16, 16)] = jnp.zeros((16,), jnp.float32)

    row0 = s * ROWS_PER_TILE
    for j in range(ROWS_PER_TILE // B2):
        pltpu.sync_copy(ob, acc.at[pl.ds(row0 + j * B2, B2)])

    plsc.subcore_barrier()

    rows = lax.iota(jnp.int32, 16)

    def compute(rbase, valf):
        r = rows + rbase
        col = lambda f: jnp.full((16,), f, jnp.int32)
        r_h = r * H
        accs = [plsc.load_gather(eb, [r_h + h]) for h in range(H)]
        for f in range(HD):
            qv = plsc.load_gather(qb, [r, col(f)])
            kv = plsc.load_gather(kb, [r, col(f)])
            accs[f // D] = accs[f // D] + qv * kv
        m = jnp.maximum(jnp.maximum(jnp.maximum(accs[0], accs[1]),
                                    jnp.maximum(accs[2], accs[3])),
                        jnp.maximum(jnp.maximum(accs[4], accs[5]),
                                    jnp.maximum(accs[6], accs[7])))
        es = [jnp.exp(a - m) for a in accs]
        tot = (((es[0] + es[1]) + (es[2] + es[3]))
               + ((es[4] + es[5]) + (es[6] + es[7])))
        sca = valf / tot
        al = [e * sca for e in es]
        for j in range(D):
            vv = plsc.load_gather(vb, [r, col(j)])
            vj = al[0] * vv
            for h in range(1, H):
                vv = plsc.load_gather(vb, [r, col(h * D + j)])
                vj = vj + al[h] * vv
            plsc.store_scatter(ob, [r, col(j)], vj)
        plsc.store_scatter(ob, [r, col(D)],
                           jnp.zeros((16,), jnp.float32) + valf)

    @pl.loop(0, BTRIPS)
    def _blk(i):
        raw = i * NW + w
        blk = jnp.minimum(raw, NBLK - 1)
        valf = (raw < NBLK).astype(jnp.float32)
        ebase = blk * B2
        pltpu.sync_copy(src_hbm.at[pl.ds(ebase, B2)], sidx)
        pltpu.sync_copy(dst_hbm.at[pl.ds(ebase, B2)], didx)
        pltpu.sync_copy(esq_hbm.at[pl.ds(ebase * H, B2 * H)], eb)
        sems = [sem0, sem1, sem2, sem3, sem4, sem5]
        svs = [sidx[pl.ds(t * 16, 16)] for t in range(B2 // 16)]
        dvs = [didx[pl.ds(t * 16, 16)] for t in range(B2 // 16)]
        cps = []
        for t in range(B2 // 16):
            sl = pl.ds(t * 16, 16)
            cps.append(pltpu.async_copy(q_hbm.at[svs[t]], qb.at[sl],
                                        sems[(3 * t) % 6]))
            cps.append(pltpu.async_copy(k_hbm.at[dvs[t]], kb.at[sl],
                                        sems[(3 * t + 1) % 6]))
            cps.append(pltpu.async_copy(v_hbm.at[dvs[t]], vb.at[sl],
                                        sems[(3 * t + 2) % 6]))
        for cp in cps:
            cp.wait()
        for t in range(B2 // 16):
            compute(t * 16, valf)
        cpo = [pltpu.async_copy(ob.at[pl.ds(t * 16, 16)], acc.at[svs[t]],
                                sems[t], add=True)
               for t in range(B2 // 16)]
        for cp in cpo:
            cp.wait()

    plsc.subcore_barrier()
    out_row = c * NP + s * ROWS_PER_TILE
    for j in range(ROWS_PER_TILE // B2):
        pltpu.sync_copy(acc.at[pl.ds(row0 + j * B2, B2)], ob)
        pltpu.sync_copy(ob, part_hbm.at[pl.ds(out_row + j * B2, B2)])


def _sc_edges(src, dst, q, k, v, esq):
    mesh = plsc.VectorSubcoreMesh(core_axis_name="c", subcore_axis_name="s")
    fn = pl.kernel(
        _sc_edges_body,
        out_type=jax.ShapeDtypeStruct((2 * NP, W), jnp.float32),
        mesh=mesh,
        compiler_params=pltpu.CompilerParams(needs_layout_passes=False),
        scratch_types=[
            pltpu.VMEM((B2,), jnp.int32),
            pltpu.VMEM((B2,), jnp.int32),
            pltpu.VMEM((B2, HD), jnp.float32),
            pltpu.VMEM((B2, HD), jnp.float32),
            pltpu.VMEM((B2, HD), jnp.float32),
            pltpu.VMEM((B2 * H,), jnp.float32),
            pltpu.VMEM((B2, W), jnp.float32),
            pltpu.VMEM_SHARED((NP, W), jnp.float32),
            pltpu.SemaphoreType.DMA,
            pltpu.SemaphoreType.DMA,
            pltpu.SemaphoreType.DMA,
            pltpu.SemaphoreType.DMA,
            pltpu.SemaphoreType.DMA,
            pltpu.SemaphoreType.DMA,
        ],
    )
    return fn(src, dst, q, k, v, esq)


def _combine_body(p0_ref, p1_ref, c0_ref, c1_ref, bo_ref, o_ref):
    csum = c0_ref[...] + c1_ref[...]
    s = p0_ref[...] + p1_ref[...]
    denom = jnp.maximum(csum, 1.0)
    o_ref[...] = s / denom + bo_ref[...] * (csum > 0.0)


def _combine(p0, p1, c0, c1, bo):
    pspec = pl.BlockSpec((N, D), lambda: (0, 0))
    cspec = pl.BlockSpec((N, 1), lambda: (0, 0))
    return pl.pallas_call(
        _combine_body,
        in_specs=[pspec, pspec, cspec, cspec,
                  pl.BlockSpec((1, D), lambda: (0, 0))],
        out_specs=pl.BlockSpec((N, D), lambda: (0, 0)),
        out_shape=jax.ShapeDtypeStruct((N, D), jnp.float32),
    )(p0, p1, c0, c1, bo.reshape(1, D))


@jax.jit
def kernel(x, edge_index, edge_attr, Wq, bq, Wk, bk, Wv, bv, We, be, Wo, bo):
    scale = 1.0 / (D ** 0.5)
    wq = Wq * scale
    bqs = bq * scale
    wv2 = jnp.einsum('fhd,hde->fhe', Wv.reshape(DF, H, D),
                     Wo.reshape(H, D, D)).reshape(DF, HD)
    bv2 = jnp.einsum('hd,hde->he', bv.reshape(H, D),
                     Wo.reshape(H, D, D)).reshape(HD)
    sel = (jnp.kron(jnp.eye(H, dtype=jnp.float32),
                    jnp.ones((D, 1), dtype=jnp.float32)) * scale)

    q, k, v = _proj(x, wq, bqs, Wk, bk, wv2, bv2)
    esq = _esq(edge_attr, We, be, sel).reshape(E * H)

    src = edge_index[0]
    dst = edge_index[1]
    part = _sc_edges(src, dst, q, k, v, esq)

    p0 = part[:N, :D]
    p1 = part[NP:NP + N, :D]
    c0 = part[:N, D:D + 1]
    c1 = part[NP:NP + N, D:D + 1]
    return _combine(p0, p1, c0, c1, bo)
